# Initial kernel scaffold; baseline (speedup 1.0000x reference)
import jax
import jax.numpy as jnp
from jax.experimental import pallas as pl


def kernel(features, edge_index, W0, b0, W1, b1, W2, b2, Wy, by, init_weight_y, tau_1, tau_2):
    raise NotImplementedError("write your pallas kernel here")


# speedup vs baseline: 7.0692x; 7.0692x over previous
import functools
import math

import jax
import jax.numpy as jnp
from jax import lax
from jax.experimental import pallas as pl
from jax.experimental.pallas import tpu as pltpu
from jax.experimental.pallas import tpu_sc as plsc

N = 10000
E = 320000
D = 128
H = 128
C = 40

NC = 2
NS = 16
NW = NC * NS
CHUNK = 128
NCHUNK = 79
EPT = NCHUNK * CHUNK
EP = NW * EPT
NP = 10112
RPT = NP // NS
TRASH = N


def _sc_mesh():
    return plsc.VectorSubcoreMesh(core_axis_name="c", subcore_axis_name="s")


def _sc_deg(dstp, zvec):
    @functools.partial(
        pl.kernel,
        out_type=jax.ShapeDtypeStruct((NC * NP,), jnp.float32),
        mesh=_sc_mesh(),
        scratch_types=[
            pltpu.VMEM((CHUNK,), jnp.int32),
            pltpu.VMEM((CHUNK,), jnp.float32),
            pltpu.VMEM((RPT,), jnp.float32),
            pltpu.VMEM_SHARED((NP,), jnp.float32),
        ],
    )
    def k(dst_hbm, z_hbm, out_hbm, didx, ones_v, zb, acc):
        cid = lax.axis_index("c")
        sid = lax.axis_index("s")
        wid = sid * NC + cid
        rbase = sid * RPT
        pltpu.sync_copy(z_hbm.at[pl.ds(rbase, RPT)], zb)
        pltpu.sync_copy(zb, acc.at[pl.ds(rbase, RPT)])
        for j in range(CHUNK // 16):
            ones_v[pl.ds(j * 16, 16)] = jnp.ones((16,), jnp.float32)
        plsc.subcore_barrier()
        ebase = wid * EPT

        def body(i, carry):
            base = ebase + i * CHUNK
            pltpu.sync_copy(dst_hbm.at[pl.ds(base, CHUNK)], didx)
            pltpu.sync_copy(ones_v, acc.at[didx], add=True)
            return carry

        lax.fori_loop(0, NCHUNK, body, 0)
        plsc.subcore_barrier()
        pltpu.sync_copy(acc.at[pl.ds(rbase, RPT)], zb)
        pltpu.sync_copy(zb, out_hbm.at[pl.ds(cid * NP + rbase, RPT)])

    return k(dstp, zvec)


def _sc_agg(hs, srcp, dstp, zmat):
    @functools.partial(
        pl.kernel,
        out_type=jax.ShapeDtypeStruct((NC * NP, H), jnp.float32),
        mesh=_sc_mesh(),
        scratch_types=[
            pltpu.VMEM((CHUNK,), jnp.int32),
            pltpu.VMEM((CHUNK,), jnp.int32),
            pltpu.VMEM((CHUNK, H), jnp.float32),
            pltpu.VMEM_SHARED((NP, H), jnp.float32),
        ],
    )
    def k(hs_hbm, src_hbm, dst_hbm, z_hbm, out_hbm, sidx, didx, rows, acc):
        cid = lax.axis_index("c")
        sid = lax.axis_index("s")
        wid = sid * NC + cid
        rbase = sid * RPT
        for r0 in range(0, RPT, CHUNK):
            nr = min(CHUNK, RPT - r0)
            pltpu.sync_copy(z_hbm.at[pl.ds(rbase + r0, nr)],
                            rows.at[pl.ds(0, nr)])
            pltpu.sync_copy(rows.at[pl.ds(0, nr)],
                            acc.at[pl.ds(rbase + r0, nr)])
        plsc.subcore_barrier()
        ebase = wid * EPT

        def body(i, carry):
            base = ebase + i * CHUNK
            pltpu.sync_copy(src_hbm.at[pl.ds(base, CHUNK)], sidx)
            pltpu.sync_copy(dst_hbm.at[pl.ds(base, CHUNK)], didx)
            pltpu.sync_copy(hs_hbm.at[sidx], rows)
            pltpu.sync_copy(rows, acc.at[didx], add=True)
            return carry

        lax.fori_loop(0, NCHUNK, body, 0)
        plsc.subcore_barrier()
        for r0 in range(0, RPT, CHUNK):
            nr = min(CHUNK, RPT - r0)
            pltpu.sync_copy(acc.at[pl.ds(rbase + r0, nr)],
                            rows.at[pl.ds(0, nr)])
            pltpu.sync_copy(rows.at[pl.ds(0, nr)],
                            out_hbm.at[pl.ds(cid * NP + rbase + r0, nr)])

    return k(hs, srcp, dstp, zmat)


_INV_LOG_C = 1.0 / math.log(float(C))


def _gate(h, wy, by, t1, t2):
    u = jnp.dot(h, wy, preferred_element_type=jnp.float32) + by
    m = jnp.max(u, axis=1, keepdims=True)
    e = jnp.exp(u - m)
    p = e / jnp.sum(e, axis=1, keepdims=True)
    ent = -jnp.sum(p * jnp.log(p + 1e-12), axis=1, keepdims=True)
    eta = ent * _INV_LOG_C
    return 0.5 * (jax.nn.sigmoid(t1 - eta) + jax.nn.sigmoid(t2 - eta))


def _pre_body(x_ref, w_ref, b_ref, d0_ref, d1_ref, hs_ref, nrm_ref):
    nrm = lax.rsqrt(jnp.maximum(d0_ref[...] + d1_ref[...], 1.0))
    t = jnp.dot(x_ref[...], w_ref[...], preferred_element_type=jnp.float32)
    hs_ref[...] = (t + b_ref[...]) * nrm
    nrm_ref[...] = nrm


def _mid1_body(p0_ref, p1_ref, w_ref, b_ref, wy_ref, by_ref, nrm_ref, tau_ref,
               h_ref, hs_ref, z_ref):
    nrm = nrm_ref[...]
    h = nrm * (p0_ref[...] + p1_ref[...])
    z = _gate(h, wy_ref[...], by_ref[...], tau_ref[0], tau_ref[1])
    t = jnp.dot(h, w_ref[...], preferred_element_type=jnp.float32)
    h_ref[...] = h
    hs_ref[...] = (t + b_ref[...]) * nrm
    z_ref[...] = z


def _mid2_body(p0_ref, p1_ref, hp_ref, zp_ref, w_ref, b_ref, wy_ref, by_ref,
               nrm_ref, tau_ref, h_ref, hs_ref, z_ref):
    nrm = nrm_ref[...]
    zp = zp_ref[...]
    agg = nrm * (p0_ref[...] + p1_ref[...])
    h = zp * agg + (1.0 - zp) * hp_ref[...]
    z = zp * _gate(h, wy_ref[...], by_ref[...], tau_ref[0], tau_ref[1])
    t = jnp.dot(h, w_ref[...], preferred_element_type=jnp.float32)
    h_ref[...] = h
    hs_ref[...] = (t + b_ref[...]) * nrm
    z_ref[...] = z


def _fin_body(p0_ref, p1_ref, hp_ref, zp_ref, wy_ref, by_ref, nrm_ref,
              out_ref):
    nrm = nrm_ref[...]
    zp = zp_ref[...]
    agg = nrm * (p0_ref[...] + p1_ref[...])
    h = zp * agg + (1.0 - zp) * hp_ref[...]
    out_ref[...] = (jnp.dot(h, wy_ref[...], preferred_element_type=jnp.float32)
                    + by_ref[...])


_R = 1000


def _row_spec():
    return pl.BlockSpec((_R, H), lambda i: (i, 0))


def _col_spec():
    return pl.BlockSpec((_R, 1), lambda i: (i, 0))


def _w_spec():
    return pl.BlockSpec((H, H), lambda i: (0, 0))


def _b_spec():
    return pl.BlockSpec((1, H), lambda i: (0, 0))


def _smem_spec():
    return pl.BlockSpec(memory_space=pltpu.SMEM)


def _rowout():
    return jax.ShapeDtypeStruct((N, H), jnp.float32)


def _colout():
    return jax.ShapeDtypeStruct((N, 1), jnp.float32)


def kernel(features, edge_index, W0, b0, W1, b1, W2, b2, Wy, by,
           init_weight_y, tau_1, tau_2):
    src = edge_index[0]
    dst = edge_index[1]
    srcp = jnp.pad(src, (0, EP - E))
    dstp = jnp.pad(dst, (0, EP - E), constant_values=TRASH)
    zvec = jnp.zeros((NP,), jnp.float32)
    zmat = jnp.zeros((NP, H), jnp.float32)
    taus = jnp.concatenate([tau_1, tau_2])
    b0r = b0.reshape(1, H)
    b1r = b1.reshape(1, H)
    b2r = b2.reshape(1, H)
    wy_m = jnp.pad(Wy, ((0, 0), (0, H - C)))
    by_m = jnp.pad(by, (0, H - C), constant_values=-1e30).reshape(1, H)
    by_z = jnp.pad(by, (0, H - C)).reshape(1, H)

    grid = (N // _R,)

    degparts = _sc_deg(dstp, zvec)
    d0 = degparts[:N].reshape(N, 1)
    d1 = degparts[NP:NP + N].reshape(N, 1)

    hs0, nrm = pl.pallas_call(
        _pre_body,
        grid=grid,
        in_specs=[_row_spec(), _w_spec(), _b_spec(), _col_spec(), _col_spec()],
        out_specs=[_row_spec(), _col_spec()],
        out_shape=[_rowout(), _colout()],
    )(features, W0, b0r, d0, d1)

    parts = _sc_agg(hs0, srcp, dstp, zmat)
    p0, p1 = parts[:N], parts[NP:NP + N]

    h1, hs1, z1 = pl.pallas_call(
        _mid1_body,
        grid=grid,
        in_specs=[_row_spec(), _row_spec(), _w_spec(), _b_spec(), _w_spec(),
                  _b_spec(), _col_spec(), _smem_spec()],
        out_specs=[_row_spec(), _row_spec(), _col_spec()],
        out_shape=[_rowout(), _rowout(), _colout()],
    )(p0, p1, W1, b1r, wy_m, by_m, nrm, taus)

    parts = _sc_agg(hs1, srcp, dstp, zmat)
    p0, p1 = parts[:N], parts[NP:NP + N]

    h2, hs2, z2 = pl.pallas_call(
        _mid2_body,
        grid=grid,
        in_specs=[_row_spec(), _row_spec(), _row_spec(), _col_spec(),
                  _w_spec(), _b_spec(), _w_spec(), _b_spec(), _col_spec(),
                  _smem_spec()],
        out_specs=[_row_spec(), _row_spec(), _col_spec()],
        out_shape=[_rowout(), _rowout(), _colout()],
    )(p0, p1, h1, z1, W2, b2r, wy_m, by_m, nrm, taus)

    parts = _sc_agg(hs2, srcp, dstp, zmat)
    p0, p1 = parts[:N], parts[NP:NP + N]

    outp = pl.pallas_call(
        _fin_body,
        grid=grid,
        in_specs=[_row_spec(), _row_spec(), _row_spec(), _col_spec(),
                  _w_spec(), _b_spec(), _col_spec()],
        out_specs=_row_spec(),
        out_shape=_rowout(),
    )(p0, p1, h2, z2, wy_m, by_z, nrm)

    output = outp[:, :C]
    all_z = jnp.concatenate([z1, z2], axis=1)
    return (output, all_z)
